# baseline (speedup 1.0000x reference)
import functools

import jax
import jax.numpy as jnp
from jax import lax
from jax.experimental import pallas as pl
from jax.experimental.pallas import tpu as pltpu
from jax.experimental.pallas import tpu_sc as plsc

_info = plsc.get_sparse_core_info()
NUM_CORES = _info.num_cores
NUM_SUBCORES = _info.num_subcores
NUM_WORKERS = NUM_CORES * NUM_SUBCORES

LANE = 128
NBUF = 4
LEAD = 3


def _make_kernel(batch: int, hist: int, dim: int):
    n_tc = batch // LANE
    tc_per_w = n_tc // NUM_WORKERS
    n_tr = dim // 8
    n_chunks = hist * tc_per_w
    mesh = plsc.VectorSubcoreMesh(core_axis_name="c", subcore_axis_name="s")

    @functools.partial(
        pl.kernel,
        mesh=mesh,
        out_type=jax.ShapeDtypeStruct((hist * n_tr, n_tc, 8 * LANE),
                                      jnp.float32),
        scratch_types=[pltpu.VMEM((tc_per_w, hist, LANE), jnp.int32)]
        + [pltpu.VMEM((LANE, dim), jnp.float32) for _ in range(NBUF)]
        + [pltpu.VMEM((n_tr * 8 * LANE,), jnp.float32) for _ in range(NBUF)]
        + [pltpu.SemaphoreType.DMA((NBUF,)), pltpu.SemaphoreType.DMA((NBUF,))],
        compiler_params=pltpu.CompilerParams(
            use_tc_tiling_on_sc=False,
            needs_layout_passes=False,
            disable_bounds_checks=True,
        ),
    )
    def gather_kernel(idx_hbm, table_hbm, out_hbm, idx_v, *rest):
        gbuf = rest[:NBUF]
        tbuf = rest[NBUF:2 * NBUF]
        gsem, osem = rest[2 * NBUF], rest[2 * NBUF + 1]
        wid = lax.axis_index("s") * NUM_CORES + lax.axis_index("c")
        tc0 = wid * tc_per_w
        pltpu.sync_copy(idx_hbm.at[pl.ds(tc0, tc_per_w)], idx_v)

        i16 = lax.iota(jnp.int32, 16)
        colstep = [i16 * LANE + hh * (16 * LANE) for hh in range(dim // 16)]

        def start_gather(c, slot):
            k, h = lax.div(c, hist), lax.rem(c, hist)
            pltpu.async_copy(table_hbm.at[idx_v.at[k, h]], gbuf[slot],
                             gsem.at[slot])

        def wait_gather(slot):
            pltpu.make_async_copy(table_hbm.at[idx_v.at[0, 0]], gbuf[slot],
                                  gsem.at[slot]).wait()

        def transpose(slot):
            for j in range(LANE):
                for hh in range(dim // 16):
                    vals = gbuf[slot][j, pl.ds(hh * 16, 16)]
                    plsc.store_scatter(tbuf[slot], [colstep[hh] + j], vals)

        def start_write(c, slot):
            k, h = lax.div(c, hist), lax.rem(c, hist)
            for tr in range(n_tr):
                pltpu.async_copy(tbuf[slot].at[pl.ds(tr * 8 * LANE, 8 * LANE)],
                                 out_hbm.at[h * n_tr + tr, tc0 + k],
                                 osem.at[slot])

        def wait_write(slot):
            for tr in range(n_tr):
                pltpu.make_async_copy(
                    tbuf[slot].at[pl.ds(tr * 8 * LANE, 8 * LANE)],
                    out_hbm.at[0, 0], osem.at[slot]).wait()

        for c in range(LEAD):
            start_gather(c, c)

        def body(go, carry):
            for b in range(NBUF):
                c = go * NBUF + b
                slot_h = (b + LEAD) % NBUF
                wait_gather(b)
                transpose(b)
                start_write(c, b)
                h2 = c + LEAD

                @pl.when(h2 < n_chunks)
                def _():
                    @pl.when(c >= NBUF - LEAD)
                    def _():
                        wait_write(slot_h)

                    start_gather(h2, slot_h)

            return carry

        lax.fori_loop(0, n_chunks // NBUF, body, 0)

        for b in range(NBUF):
            wait_write(b)

    return gather_kernel


def kernel(x, weight):
    batch, hist = x.shape
    dim = weight.shape[1]
    idx = x.astype(jnp.int32).reshape(batch // LANE, LANE, hist)
    idx = idx.transpose(0, 2, 1)
    out = _make_kernel(batch, hist, dim)(idx, weight)
    out5 = out.reshape(hist, dim // 8, batch // LANE, 8, LANE)
    return out5.transpose(2, 4, 0, 1, 3).reshape(batch, hist, dim)


# speedup vs baseline: 1.2918x; 1.2918x over previous
import functools

import jax
import jax.numpy as jnp
from jax import lax
from jax.experimental import pallas as pl
from jax.experimental.pallas import tpu as pltpu
from jax.experimental.pallas import tpu_sc as plsc

_info = plsc.get_sparse_core_info()
NUM_CORES = _info.num_cores
NUM_SUBCORES = _info.num_subcores
NUM_WORKERS = NUM_CORES * NUM_SUBCORES

LANE = 128
NBUF = 4
LEAD = 3


def _make_kernel(batch: int, hist: int, dim: int):
    n_tc = batch // LANE
    tc_per_w = n_tc // NUM_WORKERS
    n_tr = dim // 8
    n_chunks = hist * tc_per_w
    mesh = plsc.VectorSubcoreMesh(core_axis_name="c", subcore_axis_name="s")

    @functools.partial(
        pl.kernel,
        mesh=mesh,
        out_type=jax.ShapeDtypeStruct((hist * n_tr, n_tc, 8, LANE),
                                      jnp.float32),
        scratch_types=[pltpu.VMEM((tc_per_w, hist, LANE), jnp.int32)]
        + [pltpu.VMEM((LANE, dim), jnp.float32) for _ in range(NBUF)]
        + [pltpu.VMEM((dim, LANE + 1), jnp.float32) for _ in range(NBUF)]
        + [pltpu.SemaphoreType.DMA((NBUF,)), pltpu.SemaphoreType.DMA((NBUF,))],
        compiler_params=pltpu.CompilerParams(
            use_tc_tiling_on_sc=False,
            needs_layout_passes=False,
            disable_bounds_checks=True,
        ),
    )
    def gather_kernel(idx_hbm, table_hbm, out_hbm, idx_v, *rest):
        gbuf = rest[:NBUF]
        tbuf = rest[NBUF:2 * NBUF]
        gsem, osem = rest[2 * NBUF], rest[2 * NBUF + 1]
        wid = lax.axis_index("s") * NUM_CORES + lax.axis_index("c")
        tc0 = wid * tc_per_w
        pltpu.sync_copy(idx_hbm.at[pl.ds(tc0, tc_per_w)], idx_v)

        i16 = lax.iota(jnp.int32, 16)
        rowidx = [i16 + 16 * hh for hh in range(dim // 16)]

        def start_gather(c, slot):
            k, h = lax.div(c, hist), lax.rem(c, hist)
            pltpu.async_copy(table_hbm.at[idx_v.at[k, h]], gbuf[slot],
                             gsem.at[slot])

        def wait_gather(slot):
            pltpu.make_async_copy(table_hbm.at[idx_v.at[0, 0]], gbuf[slot],
                                  gsem.at[slot]).wait()

        def transpose(slot):
            for j in range(LANE):
                col = i16 * 0 + j
                for hh in range(dim // 16):
                    vals = gbuf[slot][j, pl.ds(hh * 16, 16)]
                    plsc.store_scatter(tbuf[slot], [rowidx[hh], col], vals)

        def start_write(c, slot):
            k, h = lax.div(c, hist), lax.rem(c, hist)
            for tr in range(n_tr):
                pltpu.async_copy(
                    tbuf[slot].at[pl.ds(tr * 8, 8), pl.ds(0, LANE)],
                    out_hbm.at[h * n_tr + tr, tc0 + k], osem.at[slot])

        def wait_write(slot):
            for tr in range(n_tr):
                pltpu.make_async_copy(
                    tbuf[slot].at[pl.ds(tr * 8, 8), pl.ds(0, LANE)],
                    out_hbm.at[0, 0], osem.at[slot]).wait()

        for c in range(LEAD):
            start_gather(c, c)

        def body(go, carry):
            for b in range(NBUF):
                c = go * NBUF + b
                slot_h = (b + LEAD) % NBUF
                wait_gather(b)
                transpose(b)
                start_write(c, b)
                h2 = c + LEAD

                @pl.when(h2 < n_chunks)
                def _():
                    @pl.when(c >= NBUF - LEAD)
                    def _():
                        wait_write(slot_h)

                    start_gather(h2, slot_h)

            return carry

        lax.fori_loop(0, n_chunks // NBUF, body, 0)

        for b in range(NBUF):
            wait_write(b)

    return gather_kernel


def kernel(x, weight):
    batch, hist = x.shape
    dim = weight.shape[1]
    idx = x.astype(jnp.int32).reshape(batch // LANE, LANE, hist)
    idx = idx.transpose(0, 2, 1)
    out = _make_kernel(batch, hist, dim)(idx, weight)
    out5 = out.reshape(hist, dim // 8, batch // LANE, 8, LANE)
    return out5.transpose(2, 4, 0, 1, 3).reshape(batch, hist, dim)
